# baseline (speedup 1.0000x reference)
import jax
import jax.numpy as jnp
from jax.experimental import pallas as pl
from jax.experimental.pallas import tpu as pltpu

_BM = 512
_BK = 512
_NS = 32


def _vq_kernel(zsqt_ref, z_ref, tab_ref, out_ref, rmin_ref, rpk_ref, csq_ref):
    i = pl.program_id(0)
    j = pl.program_id(1)
    nj = pl.num_programs(1)
    np_ = _BK // _NS

    tb = tab_ref[...]
    z = z_ref[...]
    zsqt = zsqt_ref[...]

    @pl.when(j == 0)
    def _init():
        rmin_ref[...] = jnp.full((_NS, _BM), jnp.inf, dtype=jnp.float32)
        rpk_ref[...] = jnp.zeros((_NS, _BM), dtype=jnp.int32)

    @pl.when(i == 0)
    def _fill_csq():
        csq_ref[pl.ds(j * _BK, _BK), :] = jnp.sum(tb * tb, axis=1, keepdims=True)

    csq = csq_ref[pl.ds(j * _BK, _BK), :]
    crosst = jax.lax.dot_general(
        tb, z, (((1,), (1,)), ((), ())),
        preferred_element_type=jnp.float32)
    dt = zsqt - 2.0 * crosst + csq
    d3 = dt.reshape(np_, _NS, _BM)

    rmin = rmin_ref[...]
    rpk = rpk_ref[...]
    base = j * np_
    for p in range(np_):
        dq = d3[p]
        upd = dq < rmin
        rmin = jnp.where(upd, dq, rmin)
        rpk = jnp.where(upd, base + p, rpk)
    rmin_ref[...] = rmin
    rpk_ref[...] = rpk

    @pl.when(j == nj - 1)
    def _finish():
        sio = jax.lax.broadcasted_iota(jnp.int32, (_NS, _BM), 0)
        v, k = rmin, rpk * _NS + sio
        s = _NS
        while s > 1:
            sh = s // 2
            va, vb = v[:sh, :], v[sh:s, :]
            ka, kb = k[:sh, :], k[sh:s, :]
            take_b = (vb < va) | ((vb == va) & (kb < ka))
            v = jnp.where(take_b, vb, va)
            k = jnp.where(take_b, kb, ka)
            s = sh
        out_ref[...] = k.reshape(_BM)


def kernel(z_e_x, table):
    B, D = z_e_x.shape
    K, _ = table.shape
    z_sq_t = jnp.sum(z_e_x * z_e_x, axis=-1)[None, :]
    return pl.pallas_call(
        _vq_kernel,
        grid=(B // _BM, K // _BK),
        in_specs=[
            pl.BlockSpec((1, _BM), lambda i, j: (0, i)),
            pl.BlockSpec((_BM, D), lambda i, j: (i, 0)),
            pl.BlockSpec((_BK, D), lambda i, j: (j, 0)),
        ],
        out_specs=pl.BlockSpec((_BM,), lambda i, j: (i,)),
        out_shape=jax.ShapeDtypeStruct((B,), jnp.int32),
        scratch_shapes=[
            pltpu.VMEM((_NS, _BM), jnp.float32),
            pltpu.VMEM((_NS, _BM), jnp.int32),
            pltpu.VMEM((K, 1), jnp.float32),
        ],
    )(z_sq_t, z_e_x, table)


# speedup vs baseline: 2.1681x; 2.1681x over previous
import jax
import jax.numpy as jnp
from jax.experimental import pallas as pl
from jax.experimental.pallas import tpu as pltpu

_BK = 512
_NS = 32


def _vq_kernel(zsqt_ref, z_ref, tab_ref, out_ref, rmin_ref, rpk_ref):
    j = pl.program_id(0)
    nj = pl.num_programs(0)
    np_ = _BK // _NS
    B = z_ref.shape[0]

    tb = tab_ref[...]
    z = z_ref[...]
    zsqt = zsqt_ref[...]

    @pl.when(j == 0)
    def _init():
        rmin_ref[...] = jnp.full((_NS, B), jnp.inf, dtype=jnp.float32)
        rpk_ref[...] = jnp.zeros((_NS, B), dtype=jnp.int32)

    csq = jnp.sum(tb * tb, axis=1, keepdims=True)
    crosst = jax.lax.dot_general(
        tb, z, (((1,), (1,)), ((), ())),
        preferred_element_type=jnp.float32)
    dt = zsqt - 2.0 * crosst + csq
    d3 = dt.reshape(np_, _NS, B)

    rmin = rmin_ref[...]
    rpk = rpk_ref[...]
    base = j * np_
    for p in range(np_):
        dq = d3[p]
        upd = dq < rmin
        rmin = jnp.where(upd, dq, rmin)
        rpk = jnp.where(upd, base + p, rpk)
    rmin_ref[...] = rmin
    rpk_ref[...] = rpk

    @pl.when(j == nj - 1)
    def _finish():
        sio = jax.lax.broadcasted_iota(jnp.int32, (_NS, B), 0)
        v, k = rmin, rpk * _NS + sio
        s = _NS
        while s > 1:
            sh = s // 2
            va, vb = v[:sh, :], v[sh:s, :]
            ka, kb = k[:sh, :], k[sh:s, :]
            take_b = (vb < va) | ((vb == va) & (kb < ka))
            v = jnp.where(take_b, vb, va)
            k = jnp.where(take_b, kb, ka)
            s = sh
        out_ref[...] = k.reshape(B)


def kernel(z_e_x, table):
    B, D = z_e_x.shape
    K, _ = table.shape
    z_sq_t = jnp.sum(z_e_x * z_e_x, axis=-1)[None, :]
    return pl.pallas_call(
        _vq_kernel,
        grid=(K // _BK,),
        in_specs=[
            pl.BlockSpec((1, B), lambda j: (0, 0)),
            pl.BlockSpec((B, D), lambda j: (0, 0)),
            pl.BlockSpec((_BK, D), lambda j: (j, 0)),
        ],
        out_specs=pl.BlockSpec((B,), lambda j: (0,)),
        out_shape=jax.ShapeDtypeStruct((B,), jnp.int32),
        scratch_shapes=[
            pltpu.VMEM((_NS, B), jnp.float32),
            pltpu.VMEM((_NS, B), jnp.int32),
        ],
    )(z_sq_t, z_e_x, table)


# speedup vs baseline: 2.5974x; 1.1980x over previous
import jax
import jax.numpy as jnp
from jax.experimental import pallas as pl

_BM = 512
_BK = 512
_NS = 32


def _vq_kernel(zsqt_ref, csqt_ref, z_ref, tab_ref, out_ref):
    z = z_ref[...]
    zsqt = zsqt_ref[...]
    K = tab_ref.shape[0]
    num_k = K // _BK
    np_ = _BK // _NS

    rmin = jnp.full((_NS, _BM), jnp.inf, dtype=jnp.float32)
    rpk = jnp.zeros((_NS, _BM), dtype=jnp.int32)
    for j in range(num_k):
        tb = tab_ref[j * _BK:(j + 1) * _BK, :]
        tb2 = tb + tb
        cross2 = jax.lax.dot_general(
            tb2, z, (((1,), (1,)), ((), ())),
            preferred_element_type=jnp.float32)
        csq = csqt_ref[j * _BK:(j + 1) * _BK, :]
        dt = zsqt - cross2 + csq
        d3 = dt.reshape(np_, _NS, _BM)
        for p in range(np_):
            dq = d3[p]
            upd = dq < rmin
            rmin = jnp.where(upd, dq, rmin)
            rpk = jnp.where(upd, jnp.int32(j * np_ + p), rpk)

    sio = jax.lax.broadcasted_iota(jnp.int32, (_NS, _BM), 0)
    v, k = rmin, rpk * _NS + sio
    s = _NS
    while s > 1:
        sh = s // 2
        va, vb = v[:sh, :], v[sh:s, :]
        ka, kb = k[:sh, :], k[sh:s, :]
        take_b = (vb < va) | ((vb == va) & (kb < ka))
        v = jnp.where(take_b, vb, va)
        k = jnp.where(take_b, kb, ka)
        s = sh
    out_ref[...] = k.reshape(_BM)


def kernel(z_e_x, table):
    B, D = z_e_x.shape
    K, _ = table.shape
    z_sq_t = jnp.sum(z_e_x * z_e_x, axis=-1)[None, :]
    c_sq_t = jnp.sum(table * table, axis=-1)[:, None]
    return pl.pallas_call(
        _vq_kernel,
        grid=(B // _BM,),
        in_specs=[
            pl.BlockSpec((1, _BM), lambda i: (0, i)),
            pl.BlockSpec((K, 1), lambda i: (0, 0)),
            pl.BlockSpec((_BM, D), lambda i: (i, 0)),
            pl.BlockSpec((K, D), lambda i: (0, 0)),
        ],
        out_specs=pl.BlockSpec((_BM,), lambda i: (i,)),
        out_shape=jax.ShapeDtypeStruct((B,), jnp.int32),
    )(z_sq_t, c_sq_t, z_e_x, table)


# speedup vs baseline: 2.7192x; 1.0469x over previous
import jax
import jax.numpy as jnp
from jax.experimental import pallas as pl

_BM = 512
_BK = 512
_NS = 32


def _vq_kernel(zsqt_ref, csqt_ref, z_ref, tab_ref, out_ref):
    z = z_ref[...]
    z2 = z + z
    zsqt = zsqt_ref[...]
    K = tab_ref.shape[0]
    num_k = K // _BK
    np_ = _BK // _NS

    rmin = jnp.full((_NS, _BM), jnp.inf, dtype=jnp.float32)
    rpk = jnp.zeros((_NS, _BM), dtype=jnp.int32)
    for j in range(num_k):
        tb = tab_ref[j * _BK:(j + 1) * _BK, :]
        cross2 = jax.lax.dot_general(
            tb, z2, (((1,), (1,)), ((), ())),
            preferred_element_type=jnp.float32)
        csq = csqt_ref[j * _BK:(j + 1) * _BK, :]
        dt = zsqt - cross2 + csq
        d3 = dt.reshape(np_, _NS, _BM)
        for p in range(np_):
            dq = d3[p]
            upd = dq < rmin
            rmin = jnp.where(upd, dq, rmin)
            rpk = jnp.where(upd, jnp.int32(j * np_ + p), rpk)

    sio = jax.lax.broadcasted_iota(jnp.int32, (_NS, _BM), 0)
    v, k = rmin, rpk * _NS + sio
    s = _NS
    while s > 1:
        sh = s // 2
        va, vb = v[:sh, :], v[sh:s, :]
        ka, kb = k[:sh, :], k[sh:s, :]
        take_b = (vb < va) | ((vb == va) & (kb < ka))
        v = jnp.where(take_b, vb, va)
        k = jnp.where(take_b, kb, ka)
        s = sh
    out_ref[...] = k.reshape(_BM)


def kernel(z_e_x, table):
    B, D = z_e_x.shape
    K, _ = table.shape
    z_sq_t = jnp.sum(z_e_x * z_e_x, axis=-1)[None, :]
    c_sq_t = jnp.sum(table * table, axis=-1)[:, None]
    return pl.pallas_call(
        _vq_kernel,
        grid=(B // _BM,),
        in_specs=[
            pl.BlockSpec((1, _BM), lambda i: (0, i)),
            pl.BlockSpec((K, 1), lambda i: (0, 0)),
            pl.BlockSpec((_BM, D), lambda i: (i, 0)),
            pl.BlockSpec((K, D), lambda i: (0, 0)),
        ],
        out_specs=pl.BlockSpec((_BM,), lambda i: (i,)),
        out_shape=jax.ShapeDtypeStruct((B,), jnp.int32),
    )(z_sq_t, c_sq_t, z_e_x, table)
